# baseline (speedup 1.0000x reference)
import jax
import jax.numpy as jnp
from jax.experimental import pallas as pl

_N = 100000
_C = 80
_BETA = 2.0
_LOSS_WEIGHT = 1.0
_BLOCK = 2000


def _qfl_kernel(x_ref, t_ref, s_ref, out_ref):
    i = pl.program_id(0)
    x = x_ref[...]
    t = t_ref[...].reshape(_BLOCK)
    s = s_ref[...].reshape(_BLOCK)

    sig = jax.nn.sigmoid(x)
    neg = (jnp.maximum(x, 0.0) + jnp.log1p(jnp.exp(-jnp.abs(x)))) * sig * sig
    row_neg = jnp.sum(neg, axis=1)

    pos_mask = (t >= 0) & (t < _C)
    tc = jnp.clip(t, 0, _C - 1)
    onehot = jax.lax.broadcasted_iota(jnp.int32, x.shape, 1) == tc[:, None]
    x_t = jnp.sum(jnp.where(onehot, x, 0.0), axis=1)
    neg_t = jnp.sum(jnp.where(onehot, neg, 0.0), axis=1)

    sig_t = jax.nn.sigmoid(x_t)
    scale = s - sig_t
    pos = (jnp.maximum(x_t, 0.0) - x_t * s
           + jnp.log1p(jnp.exp(-jnp.abs(x_t)))) * (scale * scale)

    partial = jnp.sum(row_neg + jnp.where(pos_mask, pos - neg_t, 0.0))

    @pl.when(i == 0)
    def _():
        out_ref[...] = jnp.zeros((1, 1), jnp.float32)

    out_ref[...] += partial.reshape(1, 1)


def kernel(inputs, targets, scores):
    nb = _N // _BLOCK
    t3 = targets.astype(jnp.int32).reshape(nb, 1, _BLOCK)
    s3 = scores.reshape(nb, 1, _BLOCK)
    out = pl.pallas_call(
        _qfl_kernel,
        grid=(nb,),
        in_specs=[
            pl.BlockSpec((_BLOCK, _C), lambda i: (i, 0)),
            pl.BlockSpec((1, 1, _BLOCK), lambda i: (i, 0, 0)),
            pl.BlockSpec((1, 1, _BLOCK), lambda i: (i, 0, 0)),
        ],
        out_specs=pl.BlockSpec((1, 1), lambda i: (0, 0)),
        out_shape=jax.ShapeDtypeStruct((1, 1), jnp.float32),
    )(inputs, t3, s3)
    return _LOSS_WEIGHT * (out[0, 0] / _N)


# speedup vs baseline: 1.0241x; 1.0241x over previous
import jax
import jax.numpy as jnp
from jax.experimental import pallas as pl

_N = 100000
_C = 80
_LOSS_WEIGHT = 1.0
_BLOCK = 2000


def _qfl_kernel(x_ref, t_ref, s_ref, out_ref):
    i = pl.program_id(0)
    x = x_ref[...]
    t = t_ref[...].reshape(_BLOCK, 1)
    s = s_ref[...].reshape(_BLOCK, 1)

    e = jnp.exp(-jnp.abs(x))
    l = jnp.log1p(e)
    m = jnp.maximum(x, 0.0)
    sig = jnp.where(x >= 0.0, 1.0, e) / (1.0 + e)
    neg = (m + l) * sig * sig
    d = s - sig
    pos = (m + l - x * s) * (d * d)

    tcol = jnp.where((t >= 0) & (t < _C), t, -1)
    hit = jax.lax.broadcasted_iota(jnp.int32, x.shape, 1) == tcol
    partial = jnp.sum(jnp.where(hit, pos, neg))

    @pl.when(i == 0)
    def _():
        out_ref[...] = jnp.zeros((1, 1), jnp.float32)

    out_ref[...] += partial.reshape(1, 1)


def kernel(inputs, targets, scores):
    nb = _N // _BLOCK
    t3 = targets.astype(jnp.int32).reshape(nb, _BLOCK, 1)
    s3 = scores.reshape(nb, _BLOCK, 1)
    out = pl.pallas_call(
        _qfl_kernel,
        grid=(nb,),
        in_specs=[
            pl.BlockSpec((_BLOCK, _C), lambda i: (i, 0)),
            pl.BlockSpec((1, _BLOCK, 1), lambda i: (i, 0, 0)),
            pl.BlockSpec((1, _BLOCK, 1), lambda i: (i, 0, 0)),
        ],
        out_specs=pl.BlockSpec((1, 1), lambda i: (0, 0)),
        out_shape=jax.ShapeDtypeStruct((1, 1), jnp.float32),
    )(inputs, t3, s3)
    return _LOSS_WEIGHT * (out[0, 0] / _N)


# speedup vs baseline: 2.2508x; 2.1978x over previous
import jax
import jax.numpy as jnp
from jax.experimental import pallas as pl

_N = 100000
_C = 80
_LOSS_WEIGHT = 1.0
_BLOCK = 10000


def _qfl_kernel(x_ref, t_ref, s_ref, out_ref):
    i = pl.program_id(0)
    x = x_ref[...]
    t = t_ref[...].reshape(_BLOCK)[:, None]
    s = s_ref[...].reshape(_BLOCK)[:, None]

    e = jnp.exp(-jnp.abs(x))
    l = jnp.log1p(e)
    m = jnp.maximum(x, 0.0)
    sig = jnp.where(x >= 0.0, 1.0, e) / (1.0 + e)
    neg = (m + l) * sig * sig
    d = s - sig
    pos = (m + l - x * s) * (d * d)

    tcol = jnp.where((t >= 0) & (t < _C), t, -1)
    hit = jax.lax.broadcasted_iota(jnp.int32, x.shape, 1) == tcol
    partial = jnp.sum(jnp.where(hit, pos, neg))

    @pl.when(i == 0)
    def _():
        out_ref[...] = jnp.zeros((1, 1), jnp.float32)

    out_ref[...] += partial.reshape(1, 1)


def kernel(inputs, targets, scores):
    nb = _N // _BLOCK
    t3 = targets.astype(jnp.int32).reshape(nb, 1, _BLOCK)
    s3 = scores.reshape(nb, 1, _BLOCK)
    out = pl.pallas_call(
        _qfl_kernel,
        grid=(nb,),
        in_specs=[
            pl.BlockSpec((_BLOCK, _C), lambda i: (i, 0)),
            pl.BlockSpec((1, 1, _BLOCK), lambda i: (i, 0, 0)),
            pl.BlockSpec((1, 1, _BLOCK), lambda i: (i, 0, 0)),
        ],
        out_specs=pl.BlockSpec((1, 1), lambda i: (0, 0)),
        out_shape=jax.ShapeDtypeStruct((1, 1), jnp.float32),
    )(inputs, t3, s3)
    return _LOSS_WEIGHT * (out[0, 0] / _N)


# speedup vs baseline: 2.6031x; 1.1565x over previous
import jax
import jax.numpy as jnp
from jax.experimental import pallas as pl

_N = 100000
_C = 80
_LOSS_WEIGHT = 1.0
_BLOCK = 10000


def _qfl_kernel(x_ref, t_ref, s_ref, out_ref):
    x = x_ref[...]
    t_lane = t_ref[...].reshape(1, _BLOCK)
    s_lane = s_ref[...].reshape(1, _BLOCK)
    valid = (t_lane >= 0) & (t_lane < _C)
    tcol = jnp.where(valid, t_lane, -1).reshape(_BLOCK)[:, None]
    scol = s_lane.reshape(_BLOCK)[:, None]

    e = jnp.exp(-jnp.abs(x))
    den = 1.0 + e
    sp = jnp.maximum(x, 0.0) + jnp.log(den)
    sig = jnp.where(x >= 0.0, 1.0, e) * pl.reciprocal(den, approx=True)
    neg = sp * sig * sig
    d = scol - sig
    pos = (sp - x * scol) * (d * d)
    hit = jax.lax.broadcasted_iota(jnp.int32, x.shape, 1) == tcol
    out_ref[...] = jnp.sum(jnp.where(hit, pos, neg)).reshape(1, 1, 1)


def kernel(inputs, targets, scores):
    nb = _N // _BLOCK
    t3 = targets.astype(jnp.int32).reshape(nb, 1, _BLOCK)
    s3 = scores.reshape(nb, 1, _BLOCK)
    out = pl.pallas_call(
        _qfl_kernel,
        grid=(nb,),
        in_specs=[
            pl.BlockSpec((_BLOCK, _C), lambda i: (i, 0)),
            pl.BlockSpec((1, 1, _BLOCK), lambda i: (i, 0, 0)),
            pl.BlockSpec((1, 1, _BLOCK), lambda i: (i, 0, 0)),
        ],
        out_specs=pl.BlockSpec((1, 1, 1), lambda i: (i, 0, 0)),
        out_shape=jax.ShapeDtypeStruct((nb, 1, 1), jnp.float32),
    )(inputs, t3, s3)
    return _LOSS_WEIGHT * (jnp.sum(out) / _N)


# speedup vs baseline: 2.6126x; 1.0037x over previous
import jax
import jax.numpy as jnp
from jax.experimental import pallas as pl

_N = 100000
_C = 80
_LOSS_WEIGHT = 1.0
_BLOCK = 5000


def _qfl_kernel(x_ref, t_ref, s_ref, out_ref):
    x = x_ref[...]
    t_lane = t_ref[...].reshape(1, _BLOCK)
    s_lane = s_ref[...].reshape(1, _BLOCK)
    valid = (t_lane >= 0) & (t_lane < _C)
    tcol = jnp.where(valid, t_lane, -1).reshape(_BLOCK)[:, None]
    scol = s_lane.reshape(_BLOCK)[:, None]

    e = jnp.exp(-jnp.abs(x))
    den = 1.0 + e
    sp = jnp.maximum(x, 0.0) + jnp.log(den)
    sig = jnp.where(x >= 0.0, 1.0, e) * pl.reciprocal(den, approx=True)
    neg = sp * sig * sig
    d = scol - sig
    pos = (sp - x * scol) * (d * d)
    hit = jax.lax.broadcasted_iota(jnp.int32, x.shape, 1) == tcol
    out_ref[...] = jnp.sum(jnp.where(hit, pos, neg)).reshape(1, 1, 1)


def kernel(inputs, targets, scores):
    nb = _N // _BLOCK
    t3 = targets.astype(jnp.int32).reshape(nb, 1, _BLOCK)
    s3 = scores.reshape(nb, 1, _BLOCK)
    out = pl.pallas_call(
        _qfl_kernel,
        grid=(nb,),
        in_specs=[
            pl.BlockSpec((_BLOCK, _C), lambda i: (i, 0)),
            pl.BlockSpec((1, 1, _BLOCK), lambda i: (i, 0, 0)),
            pl.BlockSpec((1, 1, _BLOCK), lambda i: (i, 0, 0)),
        ],
        out_specs=pl.BlockSpec((1, 1, 1), lambda i: (i, 0, 0)),
        out_shape=jax.ShapeDtypeStruct((nb, 1, 1), jnp.float32),
    )(inputs, t3, s3)
    return _LOSS_WEIGHT * (jnp.sum(out) / _N)


# speedup vs baseline: 5.5038x; 2.1066x over previous
import jax
import jax.numpy as jnp
from jax.experimental import pallas as pl

_N = 100000
_C = 80
_LOSS_WEIGHT = 1.0
_BC = 8


def _qfl_kernel(x_ref, t_ref, s_ref, out_ref):
    i = pl.program_id(0)
    x = x_ref[...]
    t = t_ref[...].reshape(1, _N)
    s = s_ref[...].reshape(1, _N)

    e = jnp.exp(-jnp.abs(x))
    den = 1.0 + e
    sp = jnp.maximum(x, 0.0) + jnp.log(den)
    sig = jnp.where(x >= 0.0, 1.0, e) * pl.reciprocal(den, approx=True)
    neg = sp * sig * sig
    d = s - sig
    pos = (sp - x * s) * (d * d)
    c = jax.lax.broadcasted_iota(jnp.int32, x.shape, 0) + i * _BC
    hit = t == c
    out_ref[...] = jnp.sum(jnp.where(hit, pos, neg)).reshape(1, 1, 1)


def kernel(inputs, targets, scores):
    x_t = inputs.T
    nb = _C // _BC
    out = pl.pallas_call(
        _qfl_kernel,
        grid=(nb,),
        in_specs=[
            pl.BlockSpec((_BC, _N), lambda i: (i, 0)),
            pl.BlockSpec((_N,), lambda i: (0,)),
            pl.BlockSpec((_N,), lambda i: (0,)),
        ],
        out_specs=pl.BlockSpec((1, 1, 1), lambda i: (i, 0, 0)),
        out_shape=jax.ShapeDtypeStruct((nb, 1, 1), jnp.float32),
    )(x_t, targets.astype(jnp.int32), scores)
    return _LOSS_WEIGHT * (jnp.sum(out) / _N)


# speedup vs baseline: 5.7302x; 1.0411x over previous
import jax
import jax.numpy as jnp
from jax.experimental import pallas as pl

_N = 100000
_C = 80
_LOSS_WEIGHT = 1.0
_BC = 8


def _slab_loss(x, t, s, c0):
    e = jnp.exp(-jnp.abs(x))
    den = 1.0 + e
    sp = jnp.maximum(x, 0.0) + jnp.log(den)
    sig = jnp.where(x >= 0.0, 1.0, e) * pl.reciprocal(den, approx=True)
    neg = sp * sig * sig
    d = s - sig
    pos = (sp - x * s) * (d * d)
    c = jax.lax.broadcasted_iota(jnp.int32, x.shape, 0) + c0
    return jnp.sum(jnp.where(t == c, pos, neg))


def _qfl_kernel(xa_ref, xb_ref, t_ref, s_ref, out_ref):
    i = pl.program_id(0)
    t = t_ref[...].reshape(1, _N)
    s = s_ref[...].reshape(1, _N)
    acc = _slab_loss(xa_ref[...], t, s, i * _BC)
    acc += _slab_loss(xb_ref[...], t, s, _C // 2 + i * _BC)
    out_ref[...] = acc.reshape(1, 1, 1)


def kernel(inputs, targets, scores):
    x_t = inputs.T
    nb = _C // _BC // 2
    out = pl.pallas_call(
        _qfl_kernel,
        grid=(nb,),
        in_specs=[
            pl.BlockSpec((_BC, _N), lambda i: (i, 0)),
            pl.BlockSpec((_BC, _N), lambda i: (i + _C // _BC // 2, 0)),
            pl.BlockSpec((_N,), lambda i: (0,)),
            pl.BlockSpec((_N,), lambda i: (0,)),
        ],
        out_specs=pl.BlockSpec((1, 1, 1), lambda i: (i, 0, 0)),
        out_shape=jax.ShapeDtypeStruct((nb, 1, 1), jnp.float32),
    )(x_t, x_t, targets.astype(jnp.int32), scores)
    return _LOSS_WEIGHT * (jnp.sum(out) / _N)
